# Initial kernel scaffold; baseline (speedup 1.0000x reference)
import jax
import jax.numpy as jnp
from jax.experimental import pallas as pl


def kernel(x, edge_index, edge_attr, params):
    raise NotImplementedError("write your pallas kernel here")


# speedup vs baseline: 1.9806x; 1.9806x over previous
import functools

import jax
import jax.numpy as jnp
from jax import lax
from jax.experimental import pallas as pl
from jax.experimental.pallas import tpu as pltpu
from jax.experimental.pallas import tpu_sc as plsc

_N = 10000
_E = 320000
_D = 128

_ROWS_N = 400
_ROWS_E = 2000


def _mm_body(x_ref, w_ref, b_ref, y_ref):
    y_ref[...] = jnp.dot(x_ref[...], w_ref[...],
                         preferred_element_type=jnp.float32) + b_ref[...]


def _mm(x, w, b, rows_per_tile):
    n, _ = x.shape
    dout = w.shape[1]
    grid = n // rows_per_tile
    return pl.pallas_call(
        _mm_body,
        grid=(grid,),
        in_specs=[
            pl.BlockSpec((rows_per_tile, x.shape[1]), lambda i: (i, 0)),
            pl.BlockSpec((w.shape[0], dout), lambda i: (0, 0)),
            pl.BlockSpec((1, dout), lambda i: (0, 0)),
        ],
        out_specs=pl.BlockSpec((rows_per_tile, dout), lambda i: (i, 0)),
        out_shape=jax.ShapeDtypeStruct((n, dout), jnp.float32),
    )(x, w, b.reshape(1, -1))


def _bn_mm_body(y_ref, m_ref, v_ref, g_ref, bt_ref, w_ref, b_ref, z_ref):
    yn = (y_ref[...] - m_ref[...]) / jnp.sqrt(v_ref[...] + 1e-5) * g_ref[...] \
        + bt_ref[...]
    yn = jnp.maximum(yn, 0.0)
    z_ref[...] = jnp.dot(yn, w_ref[...],
                         preferred_element_type=jnp.float32) + b_ref[...]


def _bn_mm(y, m, v, g, bt, w, b, rows_per_tile):
    n, din = y.shape
    dout = w.shape[1]
    grid = n // rows_per_tile
    return pl.pallas_call(
        _bn_mm_body,
        grid=(grid,),
        in_specs=[
            pl.BlockSpec((rows_per_tile, din), lambda i: (i, 0)),
            pl.BlockSpec((1, din), lambda i: (0, 0)),
            pl.BlockSpec((1, din), lambda i: (0, 0)),
            pl.BlockSpec((1, din), lambda i: (0, 0)),
            pl.BlockSpec((1, din), lambda i: (0, 0)),
            pl.BlockSpec((din, dout), lambda i: (0, 0)),
            pl.BlockSpec((1, dout), lambda i: (0, 0)),
        ],
        out_specs=pl.BlockSpec((rows_per_tile, dout), lambda i: (i, 0)),
        out_shape=jax.ShapeDtypeStruct((n, dout), jnp.float32),
    )(y, m.reshape(1, -1), v.reshape(1, -1), g.reshape(1, -1),
      bt.reshape(1, -1), w, b.reshape(1, -1))


def _mlp_block(x, w1, b1, g, bt, w2, b2, rows_per_tile):
    y = _mm(x, w1, b1, rows_per_tile)
    m = jnp.mean(y, axis=0)
    v = jnp.var(y, axis=0)
    return y, _bn_mm(y, m, v, g, bt, w2, b2, rows_per_tile)


def _conv_mlp_body(residual, prenorm, o0_ref, o1_ref, hin_ref, hp_ref,
                   wm1, bm1, lg, lb, wm2, bm2, gn, bn_, *out_refs):
    den = jnp.concatenate([o0_ref[:, :64], o1_ref[:, :64]], axis=-1)
    num = jnp.concatenate([o0_ref[:, 64:], o1_ref[:, 64:]], axis=-1)
    out = num / (den + 1e-16) + hin_ref[...]
    z = jnp.dot(out, wm1[...], preferred_element_type=jnp.float32) + bm1[...]
    m = jnp.mean(z, axis=-1, keepdims=True)
    zc = z - m
    v = jnp.mean(zc * zc, axis=-1, keepdims=True)
    z = zc / jnp.sqrt(v + 1e-5) * lg[...] + lb[...]
    z = jnp.maximum(z, 0.0)
    z2 = jnp.dot(z, wm2[...], preferred_element_type=jnp.float32) + bm2[...]
    hnew = hp_ref[...] + z2 if residual else z2
    out_refs[0][...] = hnew
    if prenorm:
        m2 = jnp.mean(hnew, axis=-1, keepdims=True)
        hc = hnew - m2
        v2 = jnp.mean(hc * hc, axis=-1, keepdims=True)
        r = hc / jnp.sqrt(v2 + 1e-5) * gn[...] + bn_[...]
        out_refs[1][...] = jnp.maximum(r, 0.0)


def _conv_mlp(o0, o1, hin, hprev, wm1, bm1, lg, lb, wm2, bm2, gn, bn_,
              residual, prenorm):
    rows = _ROWS_N
    grid = _N // rows
    full = pl.BlockSpec((rows, 128), lambda i: (i, 0))
    row128 = pl.BlockSpec((1, 128), lambda i: (0, 0))
    row256 = pl.BlockSpec((1, 256), lambda i: (0, 0))
    n_out = 2 if prenorm else 1
    return pl.pallas_call(
        functools.partial(_conv_mlp_body, residual, prenorm),
        grid=(grid,),
        in_specs=[full, full, full, full,
                  pl.BlockSpec((128, 256), lambda i: (0, 0)), row256, row256,
                  row256,
                  pl.BlockSpec((256, 128), lambda i: (0, 0)), row128, row128,
                  row128],
        out_specs=[full] * n_out,
        out_shape=[jax.ShapeDtypeStruct((_N, 128), jnp.float32)] * n_out,
    )(o0, o1, hin, hprev, wm1, bm1.reshape(1, -1), lg.reshape(1, -1),
      lb.reshape(1, -1), wm2, bm2.reshape(1, -1), gn.reshape(1, -1),
      bn_.reshape(1, -1))


_NTILES = 16
_EP = _E // _NTILES
_K = 80
_NCH = _EP // _K
_ZR = 624
_ZTAIL = _N - _NTILES * _ZR


def _agg_sc_build():
    mesh = plsc.VectorSubcoreMesh(core_axis_name="c", subcore_axis_name="s")
    out_t = [jax.ShapeDtypeStruct((_N, 128), jnp.float32)] * 2

    @functools.partial(
        pl.kernel, mesh=mesh, out_type=out_t,
        scratch_types=[
            pltpu.VMEM((_K,), jnp.int32),
            pltpu.VMEM((_K,), jnp.int32),
            pltpu.VMEM((_K, 128), jnp.float32),
            pltpu.VMEM((_K, 128), jnp.float32),
            pltpu.VMEM((_K, 128), jnp.float32),
            pltpu.VMEM((16,), jnp.float32),
            pltpu.VMEM_SHARED((_N, 128), jnp.float32),
            pltpu.SemaphoreType.DMA,
        ],
    )
    def k(h_hbm, ea_hbm, src_hbm, dst_hbm, t_hbm, z_hbm, out0, out1,
          src_v, dst_v, hs_v, ea_v, sc_v, t_v, acc_sp, sem):
        c_id = lax.axis_index("c")
        s_id = lax.axis_index("s")
        row0 = s_id * _ZR

        pltpu.sync_copy(z_hbm.at[pl.ds(row0, _ZR)], acc_sp.at[pl.ds(row0, _ZR)])

        @pl.when(s_id == _NTILES - 1)
        def _():
            tsl = pl.ds(_NTILES * _ZR, _ZTAIL)
            pltpu.sync_copy(z_hbm.at[tsl], acc_sp.at[tsl])

        pltpu.sync_copy(t_hbm, t_v)
        plsc.subcore_barrier()

        ebase = s_id * _EP
        c64 = c_id * 64

        def chunk(kc, carry):
            e0 = ebase + kc * _K
            pltpu.sync_copy(src_hbm.at[pl.ds(e0, _K)], src_v)
            pltpu.sync_copy(dst_hbm.at[pl.ds(e0, _K)], dst_v)
            pltpu.async_copy(h_hbm.at[src_v], hs_v, sem).wait()
            pltpu.sync_copy(ea_hbm.at[pl.ds(e0, _K)], ea_v)
            tv = t_v[...]

            def row(r, carry2):
                for j in range(4):
                    sl = pl.ds(c64 + 16 * j, 16)
                    m = jnp.maximum(hs_v[r, sl] + ea_v[r, sl], 0.0) + 1e-7
                    ex = jnp.exp(m * tv)
                    sc_v[r, pl.ds(16 * j, 16)] = ex
                    sc_v[r, pl.ds(64 + 16 * j, 16)] = m * ex
                return carry2

            lax.fori_loop(0, _K, row, 0)
            pltpu.sync_copy(sc_v, acc_sp.at[dst_v], add=True)
            return carry

        lax.fori_loop(0, _NCH, chunk, 0)
        plsc.subcore_barrier()

        sl_rows = pl.ds(row0, _ZR)
        tsl = pl.ds(_NTILES * _ZR, _ZTAIL)
        last = s_id == _NTILES - 1

        @pl.when(c_id == 0)
        def _():
            pltpu.sync_copy(acc_sp.at[sl_rows], out0.at[sl_rows])

            @pl.when(last)
            def _():
                pltpu.sync_copy(acc_sp.at[tsl], out0.at[tsl])

        @pl.when(c_id == 1)
        def _():
            pltpu.sync_copy(acc_sp.at[sl_rows], out1.at[sl_rows])

            @pl.when(last)
            def _():
                pltpu.sync_copy(acc_sp.at[tsl], out1.at[tsl])

    return k


_AGG_SC_CACHE = []


def _aggregate(h, ea, src, dst, t):
    if not _AGG_SC_CACHE:
        _AGG_SC_CACHE.append(_agg_sc_build())
    t_vec = jnp.full((16,), t, dtype=jnp.float32)
    zeros = jnp.zeros((_N, 128), dtype=jnp.float32)
    return _AGG_SC_CACHE[0](h, ea, src, dst, t_vec, zeros)


def kernel(x, edge_index, edge_attr, params):
    src = edge_index[0]
    dst = edge_index[1]

    ne = params["node_enc"]
    _, h = _mlp_block(x, ne["W1"], ne["b1"], ne["g1"], ne["bt1"], ne["W2"],
                      ne["b2"], _ROWS_N)

    ee = params["edge_enc"]
    _, ea = _mlp_block(edge_attr, ee["W1"], ee["b1"], ee["g1"], ee["bt1"],
                       ee["W2"], ee["b2"], _ROWS_E)

    c = params["convs"]
    ng = params["norms"]["g"]
    nb = params["norms"]["b"]

    r = h
    hprev = h
    for i in range(4):
        o0, o1 = _aggregate(r, ea, src, dst, c["t"][i])
        residual = i > 0
        prenorm = i < 3
        outs = _conv_mlp(o0, o1, r, hprev,
                         c["Wm1"][i], c["bm1"][i], c["lg"][i], c["lb"][i],
                         c["Wm2"][i], c["bm2"][i],
                         ng[min(i + 1, 3)], nb[min(i + 1, 3)],
                         residual, prenorm)
        if prenorm:
            hprev, r = outs
        else:
            (hprev,) = outs

    o = params["out"]
    _, z2 = _mlp_block(hprev, o["W1"], o["b1"], o["g1"], o["bt1"], o["W2"],
                       o["b2"], _ROWS_N)
    m2 = jnp.mean(z2, axis=0)
    v2 = jnp.var(z2, axis=0)
    out = _bn_mm(z2, m2, v2, o["g2"], o["bt2"], o["W3"], o["b3"], _ROWS_N)
    return out


# speedup vs baseline: 3.6410x; 1.8384x over previous
import functools

import jax
import jax.numpy as jnp
from jax import lax
from jax.experimental import pallas as pl
from jax.experimental.pallas import tpu as pltpu
from jax.experimental.pallas import tpu_sc as plsc

_N = 10000
_E = 320000
_D = 128

_ROWS_N = 400
_ROWS_E = 2000


def _mm_body(x_ref, w_ref, b_ref, y_ref):
    y_ref[...] = jnp.dot(x_ref[...], w_ref[...],
                         preferred_element_type=jnp.float32) + b_ref[...]


def _mm(x, w, b, rows_per_tile):
    n, _ = x.shape
    dout = w.shape[1]
    grid = n // rows_per_tile
    return pl.pallas_call(
        _mm_body,
        grid=(grid,),
        in_specs=[
            pl.BlockSpec((rows_per_tile, x.shape[1]), lambda i: (i, 0)),
            pl.BlockSpec((w.shape[0], dout), lambda i: (0, 0)),
            pl.BlockSpec((1, dout), lambda i: (0, 0)),
        ],
        out_specs=pl.BlockSpec((rows_per_tile, dout), lambda i: (i, 0)),
        out_shape=jax.ShapeDtypeStruct((n, dout), jnp.float32),
    )(x, w, b.reshape(1, -1))


def _bn_mm_body(y_ref, m_ref, v_ref, g_ref, bt_ref, w_ref, b_ref, z_ref):
    yn = (y_ref[...] - m_ref[...]) / jnp.sqrt(v_ref[...] + 1e-5) * g_ref[...] \
        + bt_ref[...]
    yn = jnp.maximum(yn, 0.0)
    z_ref[...] = jnp.dot(yn, w_ref[...],
                         preferred_element_type=jnp.float32) + b_ref[...]


def _bn_mm(y, m, v, g, bt, w, b, rows_per_tile):
    n, din = y.shape
    dout = w.shape[1]
    grid = n // rows_per_tile
    return pl.pallas_call(
        _bn_mm_body,
        grid=(grid,),
        in_specs=[
            pl.BlockSpec((rows_per_tile, din), lambda i: (i, 0)),
            pl.BlockSpec((1, din), lambda i: (0, 0)),
            pl.BlockSpec((1, din), lambda i: (0, 0)),
            pl.BlockSpec((1, din), lambda i: (0, 0)),
            pl.BlockSpec((1, din), lambda i: (0, 0)),
            pl.BlockSpec((din, dout), lambda i: (0, 0)),
            pl.BlockSpec((1, dout), lambda i: (0, 0)),
        ],
        out_specs=pl.BlockSpec((rows_per_tile, dout), lambda i: (i, 0)),
        out_shape=jax.ShapeDtypeStruct((n, dout), jnp.float32),
    )(y, m.reshape(1, -1), v.reshape(1, -1), g.reshape(1, -1),
      bt.reshape(1, -1), w, b.reshape(1, -1))


def _mlp_block(x, w1, b1, g, bt, w2, b2, rows_per_tile):
    y = _mm(x, w1, b1, rows_per_tile)
    m = jnp.mean(y, axis=0)
    v = jnp.var(y, axis=0)
    return y, _bn_mm(y, m, v, g, bt, w2, b2, rows_per_tile)


def _conv_mlp_body(residual, prenorm, o0_ref, o1_ref, hin_ref, hp_ref,
                   wm1, bm1, lg, lb, wm2, bm2, gn, bn_, *out_refs):
    den = jnp.concatenate([o0_ref[:, :64], o1_ref[:, :64]], axis=-1)
    num = jnp.concatenate([o0_ref[:, 64:], o1_ref[:, 64:]], axis=-1)
    out = num / (den + 1e-16) + hin_ref[...]
    z = jnp.dot(out, wm1[...], preferred_element_type=jnp.float32) + bm1[...]
    m = jnp.mean(z, axis=-1, keepdims=True)
    zc = z - m
    v = jnp.mean(zc * zc, axis=-1, keepdims=True)
    z = zc / jnp.sqrt(v + 1e-5) * lg[...] + lb[...]
    z = jnp.maximum(z, 0.0)
    z2 = jnp.dot(z, wm2[...], preferred_element_type=jnp.float32) + bm2[...]
    hnew = hp_ref[...] + z2 if residual else z2
    out_refs[0][...] = hnew
    if prenorm:
        m2 = jnp.mean(hnew, axis=-1, keepdims=True)
        hc = hnew - m2
        v2 = jnp.mean(hc * hc, axis=-1, keepdims=True)
        r = hc / jnp.sqrt(v2 + 1e-5) * gn[...] + bn_[...]
        out_refs[1][...] = jnp.maximum(r, 0.0)


def _conv_mlp(o0, o1, hin, hprev, wm1, bm1, lg, lb, wm2, bm2, gn, bn_,
              residual, prenorm):
    rows = _ROWS_N
    grid = _N // rows
    full = pl.BlockSpec((rows, 128), lambda i: (i, 0))
    row128 = pl.BlockSpec((1, 128), lambda i: (0, 0))
    row256 = pl.BlockSpec((1, 256), lambda i: (0, 0))
    n_out = 2 if prenorm else 1
    return pl.pallas_call(
        functools.partial(_conv_mlp_body, residual, prenorm),
        grid=(grid,),
        in_specs=[full, full, full, full,
                  pl.BlockSpec((128, 256), lambda i: (0, 0)), row256, row256,
                  row256,
                  pl.BlockSpec((256, 128), lambda i: (0, 0)), row128, row128,
                  row128],
        out_specs=[full] * n_out,
        out_shape=[jax.ShapeDtypeStruct((_N, 128), jnp.float32)] * n_out,
    )(o0, o1, hin, hprev, wm1, bm1.reshape(1, -1), lg.reshape(1, -1),
      lb.reshape(1, -1), wm2, bm2.reshape(1, -1), gn.reshape(1, -1),
      bn_.reshape(1, -1))


_NTILES = 16
_K = 128
_NCHT = _E // _K
_NBUF = 2
_NIT = 78
_ZR = 624
_ZTAIL = _N - _NTILES * _ZR


_HPAD = 17000


def _agg_sc_build():
    mesh = plsc.VectorSubcoreMesh(core_axis_name="c", subcore_axis_name="s")
    out_t = [jax.ShapeDtypeStruct((_HPAD, 128), jnp.float32)] * 2

    @functools.partial(
        pl.kernel, mesh=mesh, out_type=out_t,
        scratch_types=(
            [pltpu.VMEM((_K,), jnp.int32)] * (2 * _NBUF)
            + [pltpu.VMEM((_K, 128), jnp.float32)] * _NBUF
            + [pltpu.VMEM((_K * 64,), jnp.float32)] * _NBUF
            + [
                pltpu.VMEM((16,), jnp.float32),
                pltpu.VMEM_SHARED((_N, 128), jnp.float32),
                pltpu.SemaphoreType.DMA,
                pltpu.SemaphoreType.DMA,
                pltpu.SemaphoreType.DMA,
            ]
        ),
    )
    def k(h_hbm, eap_hbm, src_hbm, dst_hbm, t_hbm, z_hbm, out0, out1,
          sv0, sv1, dv0, dv1, sc0, sc1, ea0, ea1, t_v, acc_sp, gsem, esem,
          ssem):
        svs = [sv0, sv1]
        dvs = [dv0, dv1]
        scs = [sc0, sc1]
        eas = [ea0, ea1]
        c_id = lax.axis_index("c")
        s_id = lax.axis_index("s")
        row0 = s_id * _ZR

        pltpu.sync_copy(z_hbm.at[pl.ds(row0, _ZR)], acc_sp.at[pl.ds(row0, _ZR)])

        @pl.when(s_id == _NTILES - 1)
        def _():
            tsl = pl.ds(_NTILES * _ZR, _ZTAIL)
            pltpu.sync_copy(z_hbm.at[tsl], acc_sp.at[tsl])

        pltpu.sync_copy(t_hbm, t_v)
        plsc.subcore_barrier()

        c64 = c_id * 64
        ebo = c_id * (_E // 2)
        tv = t_v[...]

        def make_pair_body(scr, ear):
            def pb(pr, cz):
                a = 2 * pr
                eb = pr * 128
                for j in range(4):
                    hsl = pl.ds(c64 + 16 * j, 16)
                    ha = scr[a, hsl]
                    hb = scr[a + 1, hsl]
                    ma = jnp.maximum(ha + ear[pl.ds(eb + 16 * j, 16)],
                                     0.0) + 1e-7
                    mb = jnp.maximum(hb + ear[pl.ds(eb + 64 + 16 * j, 16)],
                                     0.0) + 1e-7
                    exa = jnp.exp(ma * tv)
                    exb = jnp.exp(mb * tv)
                    scr[a, pl.ds(16 * j, 16)] = exa
                    scr[a, pl.ds(64 + 16 * j, 16)] = ma * exa
                    scr[a + 1, pl.ds(16 * j, 16)] = exb
                    scr[a + 1, pl.ds(64 + 16 * j, 16)] = mb * exb
                return cz
            return pb

        def run_chunks(cis):
            n = len(cis)
            ih = []
            for q in range(n):
                e0 = cis[q] * _K
                ih.append(pltpu.async_copy(src_hbm.at[pl.ds(e0, _K)], svs[q],
                                           esem))
                ih.append(pltpu.async_copy(dst_hbm.at[pl.ds(e0, _K)], dvs[q],
                                           esem))
            for i in ih:
                i.wait()
            gh, eh = [], []
            for q in range(n):
                e0 = cis[q] * _K
                f0 = ebo * 128 + e0 * 64
                gh.append(pltpu.async_copy(h_hbm.at[svs[q]], scs[q], gsem))
                eh.append(pltpu.async_copy(eap_hbm.at[pl.ds(f0, _K * 64)],
                                           eas[q], esem))
            sh = []
            for q in range(n):
                gh[q].wait()
                eh[q].wait()
                lax.fori_loop(0, _K // 2, make_pair_body(scs[q], eas[q]), 0)
                sh.append(pltpu.async_copy(scs[q], acc_sp.at[dvs[q]], ssem,
                                           add=True))
            for s in sh:
                s.wait()

        def it_body(it, carry):
            base = it * _NBUF * _NTILES + s_id
            run_chunks([base + q * _NTILES for q in range(_NBUF)])
            return carry

        lax.fori_loop(0, _NIT, it_body, 0)

        @pl.when(s_id < _NCHT - _NIT * _NBUF * _NTILES)
        def _():
            run_chunks([_NIT * _NBUF * _NTILES + s_id])

        plsc.subcore_barrier()

        sl_rows = pl.ds(row0, _ZR)
        tsl = pl.ds(_NTILES * _ZR, _ZTAIL)
        last = s_id == _NTILES - 1

        @pl.when(c_id == 0)
        def _():
            pltpu.sync_copy(acc_sp.at[sl_rows], out0.at[sl_rows])

            @pl.when(last)
            def _():
                pltpu.sync_copy(acc_sp.at[tsl], out0.at[tsl])

        @pl.when(c_id == 1)
        def _():
            pltpu.sync_copy(acc_sp.at[sl_rows], out1.at[sl_rows])

            @pl.when(last)
            def _():
                pltpu.sync_copy(acc_sp.at[tsl], out1.at[tsl])

    return k


_AGG_SC_CACHE = []


def _aggregate(h, eap, src, dst, t):
    if not _AGG_SC_CACHE:
        _AGG_SC_CACHE.append(_agg_sc_build())
    t_vec = jnp.full((16,), t, dtype=jnp.float32)
    zeros = jnp.zeros((_N, 128), dtype=jnp.float32)
    hp = jnp.zeros((_HPAD, 128), dtype=jnp.float32).at[:_N].set(h)
    return _AGG_SC_CACHE[0](hp, eap, src, dst, t_vec, zeros)


def kernel(x, edge_index, edge_attr, params):
    src = edge_index[0]
    dst = edge_index[1]

    ne = params["node_enc"]
    _, h = _mlp_block(x, ne["W1"], ne["b1"], ne["g1"], ne["bt1"], ne["W2"],
                      ne["b2"], _ROWS_N)

    ee = params["edge_enc"]
    _, ea = _mlp_block(edge_attr, ee["W1"], ee["b1"], ee["g1"], ee["bt1"],
                       ee["W2"], ee["b2"], _ROWS_E)

    eap = jnp.concatenate([ea[:, :64].reshape(-1), ea[:, 64:].reshape(-1)])

    c = params["convs"]
    ng = params["norms"]["g"]
    nb = params["norms"]["b"]

    r = h
    hprev = h
    for i in range(4):
        o0, o1 = _aggregate(r, eap, src, dst, c["t"][i])
        residual = i > 0
        prenorm = i < 3
        outs = _conv_mlp(o0, o1, r, hprev,
                         c["Wm1"][i], c["bm1"][i], c["lg"][i], c["lb"][i],
                         c["Wm2"][i], c["bm2"][i],
                         ng[min(i + 1, 3)], nb[min(i + 1, 3)],
                         residual, prenorm)
        if prenorm:
            hprev, r = outs
        else:
            (hprev,) = outs

    o = params["out"]
    _, z2 = _mlp_block(hprev, o["W1"], o["b1"], o["g1"], o["bt1"], o["W2"],
                       o["b2"], _ROWS_N)
    m2 = jnp.mean(z2, axis=0)
    v2 = jnp.var(z2, axis=0)
    out = _bn_mm(z2, m2, v2, o["g2"], o["bt2"], o["W3"], o["b3"], _ROWS_N)
    return out
